# baseline (speedup 1.0000x reference)
import functools
import math

import jax
import jax.numpy as jnp
from jax import lax
from jax.experimental import pallas as pl
from jax.experimental.pallas import tpu as pltpu
from jax.experimental.pallas import tpu_sc as plsc

N = 10000
E = 320000
NFEAT = 128
NHIDDEN = 64
NCLASS = 40
NLAYERS = 4
LAMDA = 0.5
ALPHA = 0.1

NC = 2
NS = 16
NW = NC * NS
K = 128
NCH = 2560
CW = NCH // NS
EP = NCH * K
NPAD = 10240
NACC = 10112
ROWS_PER_TILE = NACC // NS
NB = 4
NH2 = NHIDDEN // 2
FH = NHIDDEN // NC
FH2 = FH // 2


def _spmm_sc_body(hpk_hbm, sd_hbm, w_hbm, out_hbm,
                  sd_v, src_v, dst_v, w_v, gbuf, sbuf, zero_v, w128,
                  acc_sh, h_sh,
                  gsem0, gsem1, gsem2, gsem3, ssem0, ssem1, ssem2, ssem3):
  cid = lax.axis_index("c")
  sid = lax.axis_index("s")
  base = sid * ROWS_PER_TILE

  evens = lax.iota(jnp.int32, 16) * 2
  odds = evens + 1

  stage_s = pltpu.async_copy(sd_hbm.at[pl.ds(sid * CW, CW)], sd_v, gsem0)
  stage_w = pltpu.async_copy(w_hbm.at[pl.ds(sid * CW, CW)], w_v, ssem0)
  stage_h = pltpu.async_copy(
      hpk_hbm.at[pl.ds(base, ROWS_PER_TILE), pl.ds(cid * FH2, FH2)],
      h_sh.at[pl.ds(base, ROWS_PER_TILE)], ssem1)

  def zero_row(r, _):
    for j in range(FH // 16):
      zero_v[r, pl.ds(j * 16, 16)] = jnp.zeros((16,), jnp.float32)
    return 0
  lax.fori_loop(0, K, zero_row, 0)
  for off in range(0, ROWS_PER_TILE, K):
    n = min(K, ROWS_PER_TILE - off)
    pltpu.sync_copy(zero_v.at[pl.ds(0, n)], acc_sh.at[pl.ds(base + off, n)])
  stage_s.wait()
  mask14 = jnp.full((16,), 16383, jnp.int32)
  def unpack_edges(q, _):
    for j in range(K // 16):
      v = sd_v[q, pl.ds(j * 16, 16)]
      src_v[q, pl.ds(j * 16, 16)] = v & mask14
      dst_v[q, pl.ds(j * 16, 16)] = lax.shift_right_logical(
          v, jnp.full((16,), 14, jnp.int32))
    return 0
  lax.fori_loop(0, CW, unpack_edges, 0)
  stage_w.wait()
  stage_h.wait()
  plsc.subcore_barrier()

  gbufs = [gbuf.at[b] for b in range(NB)]
  sbufs = [sbuf.at[b] for b in range(NB)]
  gsems = [gsem0, gsem1, gsem2, gsem3]
  ssems = [ssem0, ssem1, ssem2, ssem3]

  def scale(gb_ref, sb_ref, c):
    for j in range(K // 32):
      wa, wb = plsc.unpack(w_v[c, pl.ds(32 * j, 32)],
                           format=plsc.PackFormat.INTERLEAVED)
      plsc.store_scatter(w128, [evens + 32 * j], wa)
      plsc.store_scatter(w128, [odds + 32 * j], wb)

    def eb_body(eb, _):
      rows = [eb * 4 + k for k in range(4)]
      wvs = [plsc.load_gather(w128, [jnp.full((16,), r, jnp.int32)])
             for r in rows]
      loads = [gb_ref[r, pl.ds(0, FH2)] for r in rows]
      unp = [plsc.unpack(plsc.bitcast(v, jnp.bfloat16),
                         format=plsc.PackFormat.INTERLEAVED)
             for v in loads]
      for i, r in enumerate(rows):
        rsp = jnp.full((16,), r, jnp.int32)
        ga, gb = unp[i]
        plsc.store_scatter(sb_ref, [rsp, evens], ga * wvs[i])
        plsc.store_scatter(sb_ref, [rsp, odds], gb * wvs[i])
      return 0
    lax.fori_loop(0, K // 4, eb_body, 0)

  for b in range(NB - 1):
    pltpu.async_copy(h_sh.at[src_v.at[b]], gbufs[b], gsems[b])

  def quad_body(it, _):
    for b in range(NB):
      c = NB * it + b
      pltpu.make_async_copy(h_sh.at[src_v.at[c]], gbufs[b], gsems[b]).wait()
      bn = (b + NB - 1) % NB
      cn = jnp.minimum(c + NB - 1, CW - 1)
      pltpu.async_copy(h_sh.at[src_v.at[cn]], gbufs[bn], gsems[bn])

      @pl.when(it > 0)
      def _():
        pltpu.make_async_copy(sbufs[b], acc_sh.at[dst_v.at[c]],
                              ssems[b]).wait()
      scale(gbufs[b], sbufs[b], c)
      pltpu.async_copy(sbufs[b], acc_sh.at[dst_v.at[c]], ssems[b], add=True)
    return 0
  lax.fori_loop(0, CW // NB, quad_body, 0)
  for b in range(NB):
    pltpu.make_async_copy(sbufs[b], acc_sh.at[dst_v.at[0]], ssems[b]).wait()
  for b in range(NB - 1):
    pltpu.make_async_copy(h_sh.at[src_v.at[0]], gbufs[b], gsems[b]).wait()
  plsc.subcore_barrier()

  for off in range(0, ROWS_PER_TILE, K):
    n = min(K, ROWS_PER_TILE - off)
    pltpu.sync_copy(acc_sh.at[pl.ds(base + off, n)],
                    out_hbm.at[cid, pl.ds(base + off, n)])


@jax.jit
def _spmm_sc(hpk, sd2d, w2d):
  mesh = plsc.VectorSubcoreMesh(core_axis_name="c", subcore_axis_name="s")
  f = pl.kernel(
      _spmm_sc_body,
      out_type=jax.ShapeDtypeStruct((NC, NPAD, FH), jnp.float32),
      mesh=mesh,
      compiler_params=pltpu.CompilerParams(use_tc_tiling_on_sc=False,
                                           needs_layout_passes=False),
      scratch_types=[
          pltpu.VMEM((CW, K), jnp.int32),
          pltpu.VMEM((CW, K), jnp.int32),
          pltpu.VMEM((CW, K), jnp.int32),
          pltpu.VMEM((CW, K), jnp.bfloat16),
          pltpu.VMEM((NB, K, FH2), jnp.int32),
          pltpu.VMEM((NB, K, FH), jnp.float32),
          pltpu.VMEM((K, FH), jnp.float32),
          pltpu.VMEM((K,), jnp.float32),
          pltpu.VMEM_SHARED((NACC, FH), jnp.float32),
          pltpu.VMEM_SHARED((NACC, FH2), jnp.int32),
          pltpu.SemaphoreType.DMA,
          pltpu.SemaphoreType.DMA,
          pltpu.SemaphoreType.DMA,
          pltpu.SemaphoreType.DMA,
          pltpu.SemaphoreType.DMA,
          pltpu.SemaphoreType.DMA,
          pltpu.SemaphoreType.DMA,
          pltpu.SemaphoreType.DMA,
      ],
  )
  return f(hpk, sd2d, w2d)


def _pack_h(h):
  hb = h.astype(jnp.bfloat16).reshape(NPAD, NH2, 2)
  return jax.lax.bitcast_convert_type(hb, jnp.int32)


def _tc_input_body(x_ref, w_ref, b_ref, o_ref):
  o_ref[pl.ds(0, N)] = jnp.maximum(
      jnp.dot(x_ref[...], w_ref[...], preferred_element_type=jnp.float32)
      + b_ref[...][None, :], 0.0)


def _tc_layer_body(p_ref, h0_ref, w_ref, o_ref, *, theta):
  hi = jnp.concatenate([p_ref[0, :N], p_ref[1, :N]], axis=1)
  support = (1.0 - ALPHA) * hi + ALPHA * h0_ref[:N]
  out = theta * jnp.dot(support, w_ref[...],
                        preferred_element_type=jnp.float32) \
      + (1.0 - theta) * support
  o_ref[pl.ds(0, N)] = jnp.maximum(out, 0.0)


def _tc_final_body(h_ref, w_ref, b_ref, o_ref):
  logits = jnp.dot(h_ref[:N], w_ref[...],
                   preferred_element_type=jnp.float32) + b_ref[...][None, :]
  m = jnp.max(logits, axis=1, keepdims=True)
  shifted = logits - m
  lse = jnp.log(jnp.sum(jnp.exp(shifted), axis=1, keepdims=True))
  o_ref[...] = shifted - lse


@jax.jit
def kernel(x, edge_index, edge_weight, W0, b0, Wc, W1, b1):
  dst = edge_index[0]
  src = edge_index[1]
  pad = EP - E
  sd = src | (dst << 14)
  sd2d = jnp.concatenate([sd, jnp.zeros((pad,), jnp.int32)]).reshape(
      NCH, K)
  w2d = jnp.concatenate(
      [edge_weight, jnp.zeros((pad,), jnp.float32)]).astype(
          jnp.bfloat16).reshape(NCH, K)

  h0 = pl.pallas_call(
      _tc_input_body,
      out_shape=jax.ShapeDtypeStruct((NPAD, NHIDDEN), jnp.float32),
  )(x, W0, b0)

  layer_inner = h0
  layers_out = [h0]
  for i in range(NLAYERS):
    theta = math.log(LAMDA / (i + 1) + 1.0)
    partials = _spmm_sc(_pack_h(layer_inner), sd2d, w2d)
    layer_inner = pl.pallas_call(
        functools.partial(_tc_layer_body, theta=theta),
        out_shape=jax.ShapeDtypeStruct((NPAD, NHIDDEN), jnp.float32),
    )(partials, h0, Wc[i])
    if i % 2 == 0:
      layers_out.append(layer_inner)

  logp = pl.pallas_call(
      _tc_final_body,
      out_shape=jax.ShapeDtypeStruct((N, NCLASS), jnp.float32),
  )(layer_inner, W1, b1)
  return (logp, *[h[:N] for h in layers_out])
